# baseline (speedup 1.0000x reference)
import functools
import math

import jax
import jax.numpy as jnp
from jax import lax
from jax.experimental import pallas as pl
from jax.experimental.pallas import tpu as pltpu
from jax.experimental.pallas import tpu_sc as plsc

_NC = 2
_NS = 16
_NW = _NC * _NS
_L = 16

_G = 80
_QW = 64
_WB = 64
_NEG = float("-inf")


@functools.lru_cache(maxsize=None)
def _make_qgather(n_ast, nrp, d):
  per_w = nrp // _NW
  n_ch = per_w // _G
  mesh = plsc.VectorSubcoreMesh(core_axis_name="c", subcore_axis_name="s",
                                num_cores=_NC, num_subcores=_NS)

  def body(enc, rvp, qout, idx_v, *rest):
    rows = rest[:n_ch]
    gsems = rest[n_ch:2 * n_ch]
    wsem = rest[2 * n_ch]
    wid = lax.axis_index("s") * _NC + lax.axis_index("c")
    base = pl.multiple_of(wid * per_w, 8)
    pltpu.sync_copy(rvp.at[pl.ds(base, per_w)], idx_v)
    gds = [
        pltpu.async_copy(enc.at[idx_v.at[pl.ds(_G * c, _G)]], rows[c],
                         gsems[c]) for c in range(n_ch)
    ]
    wds = []
    for c in range(n_ch):
      gds[c].wait()
      wds.append(
          pltpu.async_copy(
              rows[c],
              qout.at[pl.ds(pl.multiple_of(base + _G * c, 8), _G)], wsem))
    for w in wds:
      w.wait()

  return pl.kernel(
      body,
      out_type=jax.ShapeDtypeStruct((nrp, d), jnp.float32),
      mesh=mesh,
      compiler_params=pltpu.CompilerParams(needs_layout_passes=False),
      scratch_types=[pltpu.VMEM((per_w,), jnp.int32)] +
      [pltpu.VMEM((_G, d), jnp.float32) for _ in range(n_ch)] +
      [pltpu.SemaphoreType.DMA for _ in range(n_ch)] +
      [pltpu.SemaphoreType.DMA],
  )


@functools.lru_cache(maxsize=None)
def _make_qk(nr, d, blk):
  inv = 1.0 / math.sqrt(d)

  def body(q_ref, wk_ref, o_ref):
    o_ref[...] = lax.dot_general(
        q_ref[...], wk_ref[...], (((1,), (1,)), ((), ())),
        preferred_element_type=jnp.float32) * inv

  return pl.pallas_call(
      body,
      grid=(nr // blk,),
      in_specs=[
          pl.BlockSpec((blk, d), lambda i: (i, 0)),
          pl.BlockSpec((d, d), lambda i: (0, 0)),
      ],
      out_specs=pl.BlockSpec((blk, d), lambda i: (i, 0)),
      out_shape=jax.ShapeDtypeStruct((nr, d), jnp.float32),
  )


@functools.lru_cache(maxsize=None)
def _make_main(n, nr, d):
  c_per_w = n // _NW
  n_chunks = c_per_w // _G
  n_pairs = (n_chunks - 1) // 2
  assert n_pairs * 2 + 1 == n_chunks
  nk = d // _L
  mesh = plsc.VectorSubcoreMesh(core_axis_name="c", subcore_axis_name="s",
                                num_cores=_NC, num_subcores=_NS)

  def body(enc, key_hbm, seg_hbm, qk_hbm, out_hbm, recs_hbm, reca_hbm,
           seg_v, key_v, rows0, rows1, qkw, nb_v, stage, rrow, rmeta, sb,
           sem0, sem1, sem2):

    def cp(src, dst):
      pltpu.async_copy(src, dst, sem2).wait()
    wid = lax.axis_index("s") * _NC + lax.axis_index("c")
    b = pl.multiple_of(wid * c_per_w, 8)
    cp(key_hbm.at[pl.ds(b, c_per_w)], key_v)
    cp(seg_hbm.at[pl.ds(b, c_per_w)], seg_v)

    cp(
        seg_hbm.at[pl.ds(pl.multiple_of(jnp.maximum(b - 16, 0), 8), 16)],
        nb_v)
    prev = jnp.where(wid == 0, jnp.int32(-1), nb_v[pl.ds(0, 16)][15])
    cp(
        seg_hbm.at[pl.ds(
            pl.multiple_of(jnp.minimum(b + c_per_w, n - 16), 8), 16)],
        nb_v)
    nxt = jnp.where(wid == _NW - 1, jnp.int32(nr + 1), nb_v[pl.ds(0, 16)][0])

    fs = seg_v[pl.ds(0, 16)][0]
    head_partial = fs == prev

    def zero_flat(ref, nwords):
      z = jnp.zeros((_L,), jnp.float32)

      def zb(t, _):
        ref[pl.ds(pl.multiple_of(t * _L, 8), _L)] = z
        return 0

      lax.fori_loop(0, nwords // _L, zb, 0)

    lane = lax.iota(jnp.int32, _L)
    zv = jnp.zeros((_L,), jnp.float32)
    rmeta[...] = jnp.where(lane == 0, jnp.float32(-1.0), 0.0)
    for k in range(nk):
      rrow[pl.ds(k * _L, _L)] = zv
    for s in range(2):
      slot = 2 * wid + s
      cp(rmeta,
                      recs_hbm.at[pl.ds(pl.multiple_of(slot * _L, 8), _L)])
      cp(rrow,
                      reca_hbm.at[pl.ds(pl.multiple_of(slot * d, 8), d)])

    zero_flat(stage, _WB * d)

    qbase0 = jnp.minimum((fs // 8) * 8, nr - _QW)
    cp(
        qk_hbm.at[pl.ds(pl.multiple_of(qbase0 * d, 8), _QW * d)], qkw)

    def load_q(s, qb):
      r = s - qb
      return tuple(
          qkw[pl.ds(pl.multiple_of(r * d + k * _L, 8), _L)]
          for k in range(nk))

    def flush_advance(rb, target):
      nf = jnp.maximum((target - rb) // _WB, 0)

      @pl.loop(0, nf)
      def _(t):
        off = pl.multiple_of((rb + t * _WB) * d, 8)
        cp(stage, out_hbm.at[pl.ds(off, _WB * d)])
        zero_flat(stage, _WB * d)

      return rb + nf * _WB

    def write_record(slot, segv, mv, lv, accs):
      meta = jnp.where(lane == 0, segv.astype(jnp.float32),
                       jnp.where(lane == 1, mv,
                                 jnp.where(lane == 2, lv, 0.0)))
      rmeta[...] = meta
      for k in range(nk):
        rrow[pl.ds(k * _L, _L)] = accs[k]
      cp(rmeta,
                      recs_hbm.at[pl.ds(pl.multiple_of(slot * _L, 8), _L)])
      cp(rrow,
                      reca_hbm.at[pl.ds(pl.multiple_of(slot * d, 8), d)])

    def stage_row(rb, segv, lv, accs):
      inv_l = 1.0 / (lv + 1e-9)
      slot = segv - rb
      for k in range(nk):
        stage[pl.ds(pl.multiple_of(slot * d + k * _L, 8), _L)] = (
            accs[k] * inv_l)

    def finalize_mid(cur, rb, mv, lv, accs):
      def rec(_):
        write_record(2 * wid, cur, mv, lv, accs)
        return 0

      def direct(_):
        stage_row(rb, cur, lv, accs)
        return 0

      lax.cond(head_partial & (cur == fs), rec, direct, 0)

    def load_state():
      mv = sb[pl.ds(0, _L)]
      lv = sb[pl.ds(_L, _L)]
      accs = tuple(sb[pl.ds(2 * _L + k * _L, _L)] for k in range(nk))
      return mv, lv, accs

    def store_state(mv, lv, accs):
      sb[pl.ds(0, _L)] = mv
      sb[pl.ds(_L, _L)] = lv
      for k in range(nk):
        sb[pl.ds(2 * _L + k * _L, _L)] = accs[k]

    def dot_row(rows_ref, j, qs):
      xs = tuple(rows_ref[j, pl.ds(k * _L, _L)] for k in range(nk))
      t01 = xs[0] * qs[0] + xs[1] * qs[1]
      t23 = xs[2] * qs[2] + xs[3] * qs[3]
      t45 = xs[4] * qs[4] + xs[5] * qs[5]
      t67 = xs[6] * qs[6] + xs[7] * qs[7]
      t = (t01 + t23) + (t45 + t67)
      return xs, jnp.full((_L,), jnp.sum(t))

    def group_body(rows_ref, jb, sv16, op):
      cur0, qb0, rb0 = op
      _cnt, last_mask = plsc.scan_count(sv16)
      nruns = plsc.all_reduce_population_count(last_mask)[0]
      neg_v = jnp.full((_L,), _NEG, jnp.float32)
      zero_v = jnp.zeros((_L,), jnp.float32)
      zeros8 = tuple(zero_v for _ in range(nk))

      def run_body(r, carry):
        cur, qb, rb, lo = carry
        lo_v = jnp.full((_L,), lo)
        sid = jnp.min(jnp.where(sv16 >= lo_v, sv16, jnp.int32(1 << 30)))

        def on_change(oper):
          cur, qb, rb = oper
          mv, lv, accs = load_state()
          finalize_mid(cur, rb, mv, lv, accs)
          rb = flush_advance(rb, sid)

          def refill(qb):
            nqb = jnp.minimum((sid // 8) * 8, nr - _QW)
            cp(
                qk_hbm.at[pl.ds(pl.multiple_of(nqb * d, 8), _QW * d)], qkw)
            return nqb

          qb = lax.cond(sid >= qb + _QW, refill, lambda q: q, qb)
          store_state(neg_v, zero_v, zeros8)
          return (sid, qb, rb)

        (cur, qb, rb) = lax.cond(sid != cur, on_change, lambda o: o,
                                 (cur, qb, rb))

        mv = sb[pl.ds(0, _L)]
        qs = load_q(cur, qb)
        maskb = sv16 == jnp.full((_L,), cur)
        maskf = maskb.astype(jnp.float32)

        xs0, s0 = dot_row(rows_ref, jb, qs)
        mf0 = jnp.full((_L,), maskf[0])
        s0e = jnp.where(mf0 > 0.5, s0, neg_v)
        m_prov = jnp.maximum(mv, s0)
        p0 = jnp.exp(s0e - m_prov)
        l_nc = p0
        acc_nc = [p0 * x for x in xs0]
        mg = s0e
        for jj in range(1, _L):
          xsj, sj = dot_row(rows_ref, jb + jj, qs)
          mfj = jnp.full((_L,), maskf[jj])
          sje = jnp.where(mfj > 0.5, sj, neg_v)
          mg = jnp.maximum(mg, sje)
          pj = jnp.exp(jnp.minimum(sje - m_prov, 60.0))
          l_nc = l_nc + pj
          acc_nc = [a + pj * x for a, x in zip(acc_nc, xsj)]
        m2 = jnp.maximum(mv, mg)
        corr = jnp.exp(mv - m2)
        cfix = jnp.exp(m_prov - m2)
        lv = sb[pl.ds(_L, _L)]
        accs = tuple(sb[pl.ds(2 * _L + k * _L, _L)] for k in range(nk))
        lv = lv * corr + cfix * l_nc
        accs = tuple(a * corr + cfix * an for a, an in zip(accs, acc_nc))
        store_state(m2, lv, accs)
        return (cur, qb, rb, sid + 1)

      out = pl.loop(0, nruns, init_carry=(cur0, qb0, rb0, cur0))(run_body)
      return (out[0], out[1], out[2])

    def issue(g, rows_ref, sem):
      off = pl.multiple_of(g * _G, 8)
      pltpu.async_copy(enc.at[key_v.at[pl.ds(off, _G)]], rows_ref, sem)

    def wait(rows_ref, sem):
      pltpu.make_async_copy(enc.at[key_v.at[pl.ds(0, _G)]], rows_ref,
                            sem).wait()

    def do_chunk(g, rows_ref, sem, st, prefetch):
      wait(rows_ref, sem)

      def gbody(u, st):
        jb = u * _L
        sv16 = seg_v[pl.ds(pl.multiple_of(g * _G + jb, 8), _L)]
        return group_body(rows_ref, jb, sv16, st)

      st = lax.fori_loop(0, _G // _L, gbody, st)
      if prefetch:
        @pl.when(g + 2 < n_chunks)
        def _():
          issue(g + 2, rows_ref, sem)
      return st

    issue(0, rows0, sem0)
    issue(1, rows1, sem1)

    store_state(jnp.full((_L,), _NEG, jnp.float32),
                jnp.zeros((_L,), jnp.float32),
                tuple(jnp.zeros((_L,), jnp.float32) for _ in range(nk)))
    rb0 = flush_advance(prev + 1, fs)
    st = (fs, qbase0, rb0)

    def pair_body(t, st):
      g = 2 * t
      st = do_chunk(g, rows0, sem0, st, True)
      st = do_chunk(g + 1, rows1, sem1, st, True)
      return st

    st = lax.fori_loop(0, n_pairs, pair_body, st)
    st = do_chunk(n_chunks - 1, rows0, sem0, st, False)

    (cur, qb, rb) = st
    mv, lv, accs = load_state()

    is_head = head_partial & (cur == fs)
    is_tail = (~is_head) & (cur == nxt)

    def f_head(_):
      write_record(2 * wid, cur, mv, lv, accs)
      return 0

    def f_tail(_):
      write_record(2 * wid + 1, cur, mv, lv, accs)
      return 0

    def f_stage(_):
      stage_row(rb, cur, lv, accs)
      return 0

    lax.cond(is_head, f_head,
             lambda o: lax.cond(is_tail, f_tail, f_stage, o), 0)

    rec_final = is_head | is_tail
    maxrow = cur - jnp.where(rec_final, 1, 0)
    end_row = jnp.where(wid == _NW - 1, jnp.int32(nr - 1), maxrow)

    rb = flush_advance(rb, end_row + 1)
    rcount = jnp.maximum(end_row + 1 - rb, 0)

    @pl.loop(0, rcount)
    def _(q):
      cp(
          stage.at[pl.ds(pl.multiple_of(q * d, 8), d)],
          out_hbm.at[pl.ds(pl.multiple_of((rb + q) * d, 8), d)])

  return pl.kernel(
      body,
      out_type=(
          jax.ShapeDtypeStruct((nr * d,), jnp.float32),
          jax.ShapeDtypeStruct((2 * _NW * _L,), jnp.float32),
          jax.ShapeDtypeStruct((2 * _NW * d,), jnp.float32),
      ),
      mesh=mesh,
      compiler_params=pltpu.CompilerParams(needs_layout_passes=False),
      scratch_types=[
          pltpu.VMEM((n // _NW,), jnp.int32),
          pltpu.VMEM((n // _NW,), jnp.int32),
          pltpu.VMEM((_G, d), jnp.float32),
          pltpu.VMEM((_G, d), jnp.float32),
          pltpu.VMEM((_QW * d,), jnp.float32),
          pltpu.VMEM((16,), jnp.int32),
          pltpu.VMEM((_WB * d,), jnp.float32),
          pltpu.VMEM((d,), jnp.float32),
          pltpu.VMEM((_L,), jnp.float32),
          pltpu.VMEM(((2 + d // _L) * _L,), jnp.float32),
          pltpu.SemaphoreType.DMA,
          pltpu.SemaphoreType.DMA,
          pltpu.SemaphoreType.DMA,
      ],
  )


@functools.lru_cache(maxsize=None)
def _make_merge(nr, d, blk):
  nrec = 2 * _NW

  def body(x_ref, meta_ref, reca_ref, wv_ref, o_ref):
    lo = (pl.program_id(0) * blk).astype(jnp.float32)
    x = x_ref[...]
    meta = meta_ref[...]
    seg = meta[:, 0:1]
    mr = meta[:, 1:2]
    lr = meta[:, 2:3]
    cols = lax.broadcasted_iota(jnp.int32, (nrec, blk), 1).astype(
        jnp.float32) + lo
    oh = jnp.broadcast_to(seg, (nrec, blk)) == cols
    ohf = oh.astype(jnp.float32)
    m_true = jnp.max(jnp.where(oh, jnp.broadcast_to(mr, (nrec, blk)),
                               _NEG), axis=0, keepdims=True)
    cov_row = jnp.max(ohf, axis=0, keepdims=True) > 0.5
    m_safe = jnp.where(cov_row, m_true, 0.0)
    w = ohf * jnp.exp(jnp.minimum(
        jnp.broadcast_to(mr, (nrec, blk)) - jnp.broadcast_to(
            m_safe, (nrec, blk)), 0.0))
    l_col = lax.dot_general(w, lr, (((0,), (0,)), ((), ())),
                            preferred_element_type=jnp.float32)
    cov_col = lax.dot_general(ohf, jnp.ones((nrec, 1), jnp.float32),
                              (((0,), (0,)), ((), ())),
                              preferred_element_type=jnp.float32) > 0.5
    acc_tot = lax.dot_general(w, reca_ref[...], (((0,), (0,)), ((), ())),
                              preferred_element_type=jnp.float32)
    merged = acc_tot / (l_col + 1e-9)
    xs = jnp.where(cov_col, merged, x)
    o_ref[...] = jnp.dot(xs, wv_ref[...], preferred_element_type=jnp.float32)

  return pl.pallas_call(
      body,
      grid=(nr // blk,),
      in_specs=[
          pl.BlockSpec((blk, d), lambda i: (i, 0)),
          pl.BlockSpec((nrec, _L), lambda i: (0, 0)),
          pl.BlockSpec((nrec, d), lambda i: (0, 0)),
          pl.BlockSpec((d, d), lambda i: (0, 0)),
      ],
      out_specs=pl.BlockSpec((blk, d), lambda i: (i, 0)),
      out_shape=jax.ShapeDtypeStruct((nr, d), jnp.float32),
  )


def kernel(ast_nodes_encodings, ast_node_to_pdg_key, ast_node_to_pdg_val,
           pdg_to_root_key, pdg_to_root_val, nr_cfg_nodes, W_k, W_v):
  enc = ast_nodes_encodings
  n_ast, d = enc.shape
  n = ast_node_to_pdg_key.shape[0]
  nr = pdg_to_root_key.shape[0]

  key = ast_node_to_pdg_key.astype(jnp.int32)
  seg = ast_node_to_pdg_val.astype(jnp.int32)
  rv = pdg_to_root_val.astype(jnp.int32)

  nrp = -(-nr // (_NW * _G)) * (_NW * _G)
  rvp = jnp.concatenate([rv, jnp.zeros((nrp - nr,), jnp.int32)])

  blk = 2000 if nr % 2000 == 0 else nr
  q = _make_qgather(n_ast, nrp, d)(enc, rvp)[:nr]
  qk = _make_qk(nr, d, blk)(q, W_k)
  out_pre, recs, reca = _make_main(n, nr, d)(
      enc, key, seg, jnp.reshape(qk, (nr * d,)))
  return _make_merge(nr, d, blk)(
      jnp.reshape(out_pre, (nr, d)), jnp.reshape(recs, (2 * _NW, _L)),
      jnp.reshape(reca, (2 * _NW, d)), W_v)


# speedup vs baseline: 1.0035x; 1.0035x over previous
import functools
import math

import jax
import jax.numpy as jnp
from jax import lax
from jax.experimental import pallas as pl
from jax.experimental.pallas import tpu as pltpu
from jax.experimental.pallas import tpu_sc as plsc

_NC = 2
_NS = 16
_NW = _NC * _NS
_L = 16

_G = 80
_QW = 128
_WB = 64
_NEG = float("-inf")


@functools.lru_cache(maxsize=None)
def _make_qgather(n_ast, nrp, d):
  per_w = nrp // _NW
  n_ch = per_w // _G
  mesh = plsc.VectorSubcoreMesh(core_axis_name="c", subcore_axis_name="s",
                                num_cores=_NC, num_subcores=_NS)

  def body(enc, rvp, qout, idx_v, *rest):
    rows = rest[:n_ch]
    gsems = rest[n_ch:2 * n_ch]
    wsem = rest[2 * n_ch]
    wid = lax.axis_index("s") * _NC + lax.axis_index("c")
    base = pl.multiple_of(wid * per_w, 8)
    pltpu.sync_copy(rvp.at[pl.ds(base, per_w)], idx_v)
    gds = [
        pltpu.async_copy(enc.at[idx_v.at[pl.ds(_G * c, _G)]], rows[c],
                         gsems[c]) for c in range(n_ch)
    ]
    wds = []
    for c in range(n_ch):
      gds[c].wait()
      wds.append(
          pltpu.async_copy(
              rows[c],
              qout.at[pl.ds(pl.multiple_of(base + _G * c, 8), _G)], wsem))
    for w in wds:
      w.wait()

  return pl.kernel(
      body,
      out_type=jax.ShapeDtypeStruct((nrp, d), jnp.float32),
      mesh=mesh,
      compiler_params=pltpu.CompilerParams(needs_layout_passes=False),
      scratch_types=[pltpu.VMEM((per_w,), jnp.int32)] +
      [pltpu.VMEM((_G, d), jnp.float32) for _ in range(n_ch)] +
      [pltpu.SemaphoreType.DMA for _ in range(n_ch)] +
      [pltpu.SemaphoreType.DMA],
  )


@functools.lru_cache(maxsize=None)
def _make_qk(nr, d, blk):
  inv = 1.0 / math.sqrt(d)

  def body(q_ref, wk_ref, o_ref):
    o_ref[...] = lax.dot_general(
        q_ref[...], wk_ref[...], (((1,), (1,)), ((), ())),
        preferred_element_type=jnp.float32) * inv

  return pl.pallas_call(
      body,
      grid=(nr // blk,),
      in_specs=[
          pl.BlockSpec((blk, d), lambda i: (i, 0)),
          pl.BlockSpec((d, d), lambda i: (0, 0)),
      ],
      out_specs=pl.BlockSpec((blk, d), lambda i: (i, 0)),
      out_shape=jax.ShapeDtypeStruct((nr, d), jnp.float32),
  )


@functools.lru_cache(maxsize=None)
def _make_main(n, nr, d):
  c_per_w = n // _NW
  n_chunks = c_per_w // _G
  n_pairs = (n_chunks - 1) // 2
  assert n_pairs * 2 + 1 == n_chunks
  nk = d // _L
  mesh = plsc.VectorSubcoreMesh(core_axis_name="c", subcore_axis_name="s",
                                num_cores=_NC, num_subcores=_NS)

  def body(enc, key_hbm, seg_hbm, qk_hbm, out_hbm, recs_hbm, reca_hbm,
           seg_v, key_v, rows0, rows1, qkw, nb_v, stage, rrow, rmeta, sb,
           sem0, sem1, sem2):

    def cp(src, dst):
      pltpu.async_copy(src, dst, sem2).wait()
    wid = lax.axis_index("s") * _NC + lax.axis_index("c")
    b = pl.multiple_of(wid * c_per_w, 8)
    cp(key_hbm.at[pl.ds(b, c_per_w)], key_v)
    cp(seg_hbm.at[pl.ds(b, c_per_w)], seg_v)

    cp(
        seg_hbm.at[pl.ds(pl.multiple_of(jnp.maximum(b - 16, 0), 8), 16)],
        nb_v)
    prev = jnp.where(wid == 0, jnp.int32(-1), nb_v[pl.ds(0, 16)][15])
    cp(
        seg_hbm.at[pl.ds(
            pl.multiple_of(jnp.minimum(b + c_per_w, n - 16), 8), 16)],
        nb_v)
    nxt = jnp.where(wid == _NW - 1, jnp.int32(nr + 1), nb_v[pl.ds(0, 16)][0])

    fs = seg_v[pl.ds(0, 16)][0]
    head_partial = fs == prev

    def zero_flat(ref, nwords):
      z = jnp.zeros((_L,), jnp.float32)

      def zb(t, _):
        ref[pl.ds(pl.multiple_of(t * _L, 8), _L)] = z
        return 0

      lax.fori_loop(0, nwords // _L, zb, 0)

    lane = lax.iota(jnp.int32, _L)
    zv = jnp.zeros((_L,), jnp.float32)
    rmeta[...] = jnp.where(lane == 0, jnp.float32(-1.0), 0.0)
    for k in range(nk):
      rrow[pl.ds(k * _L, _L)] = zv
    for s in range(2):
      slot = 2 * wid + s
      cp(rmeta,
                      recs_hbm.at[pl.ds(pl.multiple_of(slot * _L, 8), _L)])
      cp(rrow,
                      reca_hbm.at[pl.ds(pl.multiple_of(slot * d, 8), d)])

    zero_flat(stage, _WB * d)

    qbase0 = jnp.minimum((fs // 8) * 8, nr - _QW)
    cp(
        qk_hbm.at[pl.ds(pl.multiple_of(qbase0 * d, 8), _QW * d)], qkw)

    def load_q(s, qb):
      r = s - qb
      return tuple(
          qkw[pl.ds(pl.multiple_of(r * d + k * _L, 8), _L)]
          for k in range(nk))

    def flush_advance(rb, target):
      nf = jnp.maximum((target - rb) // _WB, 0)

      @pl.loop(0, nf)
      def _(t):
        off = pl.multiple_of((rb + t * _WB) * d, 8)
        cp(stage, out_hbm.at[pl.ds(off, _WB * d)])
        zero_flat(stage, _WB * d)

      return rb + nf * _WB

    def write_record(slot, segv, mv, lv, accs):
      meta = jnp.where(lane == 0, segv.astype(jnp.float32),
                       jnp.where(lane == 1, mv,
                                 jnp.where(lane == 2, lv, 0.0)))
      rmeta[...] = meta
      for k in range(nk):
        rrow[pl.ds(k * _L, _L)] = accs[k]
      cp(rmeta,
                      recs_hbm.at[pl.ds(pl.multiple_of(slot * _L, 8), _L)])
      cp(rrow,
                      reca_hbm.at[pl.ds(pl.multiple_of(slot * d, 8), d)])

    def stage_row(rb, segv, lv, accs):
      inv_l = 1.0 / (lv + 1e-9)
      slot = segv - rb
      for k in range(nk):
        stage[pl.ds(pl.multiple_of(slot * d + k * _L, 8), _L)] = (
            accs[k] * inv_l)

    def finalize_mid(cur, rb, mv, lv, accs):
      def rec(_):
        write_record(2 * wid, cur, mv, lv, accs)
        return 0

      def direct(_):
        stage_row(rb, cur, lv, accs)
        return 0

      lax.cond(head_partial & (cur == fs), rec, direct, 0)

    def load_state():
      mv = sb[pl.ds(0, _L)]
      lv = sb[pl.ds(_L, _L)]
      accs = tuple(sb[pl.ds(2 * _L + k * _L, _L)] for k in range(nk))
      return mv, lv, accs

    def store_state(mv, lv, accs):
      sb[pl.ds(0, _L)] = mv
      sb[pl.ds(_L, _L)] = lv
      for k in range(nk):
        sb[pl.ds(2 * _L + k * _L, _L)] = accs[k]

    def dot_row(rows_ref, j, qs):
      xs = tuple(rows_ref[j, pl.ds(k * _L, _L)] for k in range(nk))
      t01 = xs[0] * qs[0] + xs[1] * qs[1]
      t23 = xs[2] * qs[2] + xs[3] * qs[3]
      t45 = xs[4] * qs[4] + xs[5] * qs[5]
      t67 = xs[6] * qs[6] + xs[7] * qs[7]
      t = (t01 + t23) + (t45 + t67)
      return xs, jnp.full((_L,), jnp.sum(t))

    def group_body(rows_ref, jb, sv16, op):
      cur0, qb0, rb0 = op
      _cnt, last_mask = plsc.scan_count(sv16)
      nruns = plsc.all_reduce_population_count(last_mask)[0]
      neg_v = jnp.full((_L,), _NEG, jnp.float32)
      zero_v = jnp.zeros((_L,), jnp.float32)
      zeros8 = tuple(zero_v for _ in range(nk))

      def run_body(r, carry):
        cur, qb, rb, lo = carry
        lo_v = jnp.full((_L,), lo)
        sid = jnp.min(jnp.where(sv16 >= lo_v, sv16, jnp.int32(1 << 30)))

        def on_change(oper):
          cur, qb, rb = oper
          mv, lv, accs = load_state()
          finalize_mid(cur, rb, mv, lv, accs)
          rb = flush_advance(rb, sid)

          def refill(qb):
            nqb = jnp.minimum((sid // 8) * 8, nr - _QW)
            cp(
                qk_hbm.at[pl.ds(pl.multiple_of(nqb * d, 8), _QW * d)], qkw)
            return nqb

          qb = lax.cond(sid >= qb + _QW, refill, lambda q: q, qb)
          store_state(neg_v, zero_v, zeros8)
          return (sid, qb, rb)

        (cur, qb, rb) = lax.cond(sid != cur, on_change, lambda o: o,
                                 (cur, qb, rb))

        mv = sb[pl.ds(0, _L)]
        qs = load_q(cur, qb)
        maskb = sv16 == jnp.full((_L,), cur)
        maskf = maskb.astype(jnp.float32)

        xs0, s0 = dot_row(rows_ref, jb, qs)
        mf0 = jnp.full((_L,), maskf[0])
        s0e = jnp.where(mf0 > 0.5, s0, neg_v)
        m_prov = jnp.maximum(mv, s0)
        p0 = jnp.exp(s0e - m_prov)
        l_nc = p0
        acc_nc = [p0 * x for x in xs0]
        mg = s0e
        for jj in range(1, _L):
          xsj, sj = dot_row(rows_ref, jb + jj, qs)
          mfj = jnp.full((_L,), maskf[jj])
          sje = jnp.where(mfj > 0.5, sj, neg_v)
          mg = jnp.maximum(mg, sje)
          pj = jnp.exp(jnp.minimum(sje - m_prov, 60.0))
          l_nc = l_nc + pj
          acc_nc = [a + pj * x for a, x in zip(acc_nc, xsj)]
        m2 = jnp.maximum(mv, mg)
        corr = jnp.exp(mv - m2)
        cfix = jnp.exp(m_prov - m2)
        lv = sb[pl.ds(_L, _L)]
        accs = tuple(sb[pl.ds(2 * _L + k * _L, _L)] for k in range(nk))
        lv = lv * corr + cfix * l_nc
        accs = tuple(a * corr + cfix * an for a, an in zip(accs, acc_nc))
        store_state(m2, lv, accs)
        return (cur, qb, rb, sid + 1)

      out = pl.loop(0, nruns, init_carry=(cur0, qb0, rb0, cur0))(run_body)
      return (out[0], out[1], out[2])

    def issue(g, rows_ref, sem):
      off = pl.multiple_of(g * _G, 8)
      pltpu.async_copy(enc.at[key_v.at[pl.ds(off, _G)]], rows_ref, sem)

    def wait(rows_ref, sem):
      pltpu.make_async_copy(enc.at[key_v.at[pl.ds(0, _G)]], rows_ref,
                            sem).wait()

    def do_chunk(g, rows_ref, sem, st, prefetch):
      wait(rows_ref, sem)

      def gbody(u, st):
        jb = u * _L
        sv16 = seg_v[pl.ds(pl.multiple_of(g * _G + jb, 8), _L)]
        return group_body(rows_ref, jb, sv16, st)

      st = lax.fori_loop(0, _G // _L, gbody, st)
      if prefetch:
        @pl.when(g + 2 < n_chunks)
        def _():
          issue(g + 2, rows_ref, sem)
      return st

    issue(0, rows0, sem0)
    issue(1, rows1, sem1)

    store_state(jnp.full((_L,), _NEG, jnp.float32),
                jnp.zeros((_L,), jnp.float32),
                tuple(jnp.zeros((_L,), jnp.float32) for _ in range(nk)))
    rb0 = flush_advance(prev + 1, fs)
    st = (fs, qbase0, rb0)

    def pair_body(t, st):
      g = 2 * t
      st = do_chunk(g, rows0, sem0, st, True)
      st = do_chunk(g + 1, rows1, sem1, st, True)
      return st

    st = lax.fori_loop(0, n_pairs, pair_body, st)
    st = do_chunk(n_chunks - 1, rows0, sem0, st, False)

    (cur, qb, rb) = st
    mv, lv, accs = load_state()

    is_head = head_partial & (cur == fs)
    is_tail = (~is_head) & (cur == nxt)

    def f_head(_):
      write_record(2 * wid, cur, mv, lv, accs)
      return 0

    def f_tail(_):
      write_record(2 * wid + 1, cur, mv, lv, accs)
      return 0

    def f_stage(_):
      stage_row(rb, cur, lv, accs)
      return 0

    lax.cond(is_head, f_head,
             lambda o: lax.cond(is_tail, f_tail, f_stage, o), 0)

    rec_final = is_head | is_tail
    maxrow = cur - jnp.where(rec_final, 1, 0)
    end_row = jnp.where(wid == _NW - 1, jnp.int32(nr - 1), maxrow)

    rb = flush_advance(rb, end_row + 1)
    rcount = jnp.maximum(end_row + 1 - rb, 0)

    @pl.loop(0, rcount)
    def _(q):
      cp(
          stage.at[pl.ds(pl.multiple_of(q * d, 8), d)],
          out_hbm.at[pl.ds(pl.multiple_of((rb + q) * d, 8), d)])

  return pl.kernel(
      body,
      out_type=(
          jax.ShapeDtypeStruct((nr * d,), jnp.float32),
          jax.ShapeDtypeStruct((2 * _NW * _L,), jnp.float32),
          jax.ShapeDtypeStruct((2 * _NW * d,), jnp.float32),
      ),
      mesh=mesh,
      compiler_params=pltpu.CompilerParams(needs_layout_passes=False),
      scratch_types=[
          pltpu.VMEM((n // _NW,), jnp.int32),
          pltpu.VMEM((n // _NW,), jnp.int32),
          pltpu.VMEM((_G, d), jnp.float32),
          pltpu.VMEM((_G, d), jnp.float32),
          pltpu.VMEM((_QW * d,), jnp.float32),
          pltpu.VMEM((16,), jnp.int32),
          pltpu.VMEM((_WB * d,), jnp.float32),
          pltpu.VMEM((d,), jnp.float32),
          pltpu.VMEM((_L,), jnp.float32),
          pltpu.VMEM(((2 + d // _L) * _L,), jnp.float32),
          pltpu.SemaphoreType.DMA,
          pltpu.SemaphoreType.DMA,
          pltpu.SemaphoreType.DMA,
      ],
  )


@functools.lru_cache(maxsize=None)
def _make_merge(nr, d, blk):
  nrec = 2 * _NW

  def body(x_ref, meta_ref, reca_ref, wv_ref, o_ref):
    lo = (pl.program_id(0) * blk).astype(jnp.float32)
    x = x_ref[...]
    meta = meta_ref[...]
    seg = meta[:, 0:1]
    mr = meta[:, 1:2]
    lr = meta[:, 2:3]
    cols = lax.broadcasted_iota(jnp.int32, (nrec, blk), 1).astype(
        jnp.float32) + lo
    oh = jnp.broadcast_to(seg, (nrec, blk)) == cols
    ohf = oh.astype(jnp.float32)
    m_true = jnp.max(jnp.where(oh, jnp.broadcast_to(mr, (nrec, blk)),
                               _NEG), axis=0, keepdims=True)
    cov_row = jnp.max(ohf, axis=0, keepdims=True) > 0.5
    m_safe = jnp.where(cov_row, m_true, 0.0)
    w = ohf * jnp.exp(jnp.minimum(
        jnp.broadcast_to(mr, (nrec, blk)) - jnp.broadcast_to(
            m_safe, (nrec, blk)), 0.0))
    l_col = lax.dot_general(w, lr, (((0,), (0,)), ((), ())),
                            preferred_element_type=jnp.float32)
    cov_col = lax.dot_general(ohf, jnp.ones((nrec, 1), jnp.float32),
                              (((0,), (0,)), ((), ())),
                              preferred_element_type=jnp.float32) > 0.5
    acc_tot = lax.dot_general(w, reca_ref[...], (((0,), (0,)), ((), ())),
                              preferred_element_type=jnp.float32)
    merged = acc_tot / (l_col + 1e-9)
    xs = jnp.where(cov_col, merged, x)
    o_ref[...] = jnp.dot(xs, wv_ref[...], preferred_element_type=jnp.float32)

  return pl.pallas_call(
      body,
      grid=(nr // blk,),
      in_specs=[
          pl.BlockSpec((blk, d), lambda i: (i, 0)),
          pl.BlockSpec((nrec, _L), lambda i: (0, 0)),
          pl.BlockSpec((nrec, d), lambda i: (0, 0)),
          pl.BlockSpec((d, d), lambda i: (0, 0)),
      ],
      out_specs=pl.BlockSpec((blk, d), lambda i: (i, 0)),
      out_shape=jax.ShapeDtypeStruct((nr, d), jnp.float32),
  )


def kernel(ast_nodes_encodings, ast_node_to_pdg_key, ast_node_to_pdg_val,
           pdg_to_root_key, pdg_to_root_val, nr_cfg_nodes, W_k, W_v):
  enc = ast_nodes_encodings
  n_ast, d = enc.shape
  n = ast_node_to_pdg_key.shape[0]
  nr = pdg_to_root_key.shape[0]

  key = ast_node_to_pdg_key.astype(jnp.int32)
  seg = ast_node_to_pdg_val.astype(jnp.int32)
  rv = pdg_to_root_val.astype(jnp.int32)

  nrp = -(-nr // (_NW * _G)) * (_NW * _G)
  rvp = jnp.concatenate([rv, jnp.zeros((nrp - nr,), jnp.int32)])

  blk = 2000 if nr % 2000 == 0 else nr
  q = _make_qgather(n_ast, nrp, d)(enc, rvp)[:nr]
  qk = _make_qk(nr, d, blk)(q, W_k)
  out_pre, recs, reca = _make_main(n, nr, d)(
      enc, key, seg, jnp.reshape(qk, (nr * d,)))
  return _make_merge(nr, d, blk)(
      jnp.reshape(out_pre, (nr, d)), jnp.reshape(recs, (2 * _NW, _L)),
      jnp.reshape(reca, (2 * _NW, d)), W_v)
